# baseline (speedup 1.0000x reference)
import functools

import jax
import jax.numpy as jnp
from jax import lax
from jax.experimental import pallas as pl
from jax.experimental.pallas import tpu as pltpu
from jax.experimental.pallas import tpu_sc as plsc

L = 128
N = L * L
NUM_CORES = 2
NUM_SUBCORES = 16
NUM_WORKERS = NUM_CORES * NUM_SUBCORES
LANES = 16
CHUNKS = L // LANES
UNROLL = 4
DEPTH = 3


def _make_sc_permute(batch):
    assert batch % (2 * NUM_WORKERS) == 0
    rows_per_worker = batch // NUM_WORKERS

    mesh = plsc.VectorSubcoreMesh(
        core_axis_name="c",
        subcore_axis_name="s",
        num_cores=NUM_CORES,
        num_subcores=NUM_SUBCORES,
    )

    @functools.partial(
        pl.kernel,
        out_type=jax.ShapeDtypeStruct((batch, 1, L, L), jnp.float32),
        mesh=mesh,
        scratch_types=[
            pltpu.VMEM((N,), jnp.float32),
            pltpu.VMEM((N,), jnp.float32),
            pltpu.VMEM((N,), jnp.float32),
            pltpu.VMEM((1, L, L), jnp.float32),
            pltpu.VMEM((1, L, L), jnp.float32),
            pltpu.VMEM((1, L, L), jnp.float32),
            pltpu.SemaphoreType.DMA,
            pltpu.SemaphoreType.DMA,
            pltpu.SemaphoreType.DMA,
            pltpu.SemaphoreType.DMA,
            pltpu.SemaphoreType.DMA,
            pltpu.SemaphoreType.DMA,
        ],
        compiler_params=pltpu.CompilerParams(needs_layout_passes=False),
    )
    def permute(x_hbm, out_hbm, xin0_v, xin1_v, xin2_v,
                out0_v, out1_v, out2_v, in_sem0, in_sem1, in_sem2,
                out_sem0, out_sem1, out_sem2):
        wid = lax.axis_index("s") * NUM_CORES + lax.axis_index("c")
        base = wid * rows_per_worker
        xin_bufs = (xin0_v, xin1_v, xin2_v)
        out_bufs = (out0_v, out1_v, out2_v)
        in_sems = (in_sem0, in_sem1, in_sem2)
        out_sems = (out_sem0, out_sem1, out_sem2)

        H = N // 2

        def start_in(row, b):
            pltpu.async_copy(x_hbm.at[row, pl.ds(0, H)],
                             xin_bufs[b].at[pl.ds(0, H)], in_sems[b])
            pltpu.async_copy(x_hbm.at[row, pl.ds(H, H)],
                             xin_bufs[b].at[pl.ds(H, H)], in_sems[b])

        def wait_in(row, b):
            pltpu.make_async_copy(x_hbm.at[row, pl.ds(0, H)],
                                  xin_bufs[b].at[pl.ds(0, H)],
                                  in_sems[b]).wait()
            pltpu.make_async_copy(x_hbm.at[row, pl.ds(H, H)],
                                  xin_bufs[b].at[pl.ds(H, H)],
                                  in_sems[b]).wait()

        for b in range(DEPTH):
            start_in(base + b, b)

        ii = lax.iota(jnp.int32, LANES)
        col129 = tuple(129 * (u * LANES) + 129 * ii for u in range(CHUNKS))
        thresh = tuple(N + u * LANES + ii for u in range(CHUNKS))

        def do_gather(b):
            @plsc.parallel_loop(0, L, step=1, unroll=UNROLL)
            def _gather(r):
                rbase = r * L
                for u in range(CHUNKS):
                    flat = rbase + col129[u]
                    idx = jnp.where(flat >= thresh[u], flat - N, flat)
                    out_bufs[b][0, r, pl.ds(u * LANES, LANES)] = (
                        plsc.load_gather(xin_bufs[b], [idx]))

        groups = rows_per_worker // DEPTH
        rem = rows_per_worker % DEPTH

        def group_body(g, carry):
            for b in range(DEPTH):
                row = base + g * DEPTH + b
                wait_in(row, b)
                @pl.when(g > 0)
                def _wait_out():
                    pltpu.make_async_copy(
                        out_bufs[b], out_hbm.at[row - DEPTH],
                        out_sems[b]).wait()
                do_gather(b)
                pltpu.async_copy(out_bufs[b], out_hbm.at[row], out_sems[b])

                @pl.when(row + DEPTH < base + rows_per_worker)
                def _refill():
                    start_in(row + DEPTH, b)
            return carry

        lax.fori_loop(0, groups, group_body, 0, unroll=False)

        for j in range(rem):
            b = j
            row = base + groups * DEPTH + j
            wait_in(row, b)
            pltpu.make_async_copy(
                out_bufs[b], out_hbm.at[row - DEPTH], out_sems[b]).wait()
            do_gather(b)
            pltpu.async_copy(out_bufs[b], out_hbm.at[row], out_sems[b])

        for k in range(rows_per_worker - DEPTH, rows_per_worker):
            b = k % DEPTH
            pltpu.make_async_copy(
                out_bufs[b], out_hbm.at[base + k], out_sems[b]).wait()

    return permute


def kernel(x):
    batch = x.shape[0]
    x = x.reshape(batch, N)
    return _make_sc_permute(batch)(x)
